# Initial kernel scaffold; baseline (speedup 1.0000x reference)
import jax
import jax.numpy as jnp
from jax.experimental import pallas as pl


def kernel(x_left, x_right, edge_index_left, edge_index_right, node2graph_left, node2graph_right, W1, b1, W2, b2, Wf, bf):
    raise NotImplementedError("write your pallas kernel here")


# speedup vs baseline: 35.8155x; 35.8155x over previous
import functools

import jax
import jax.numpy as jnp
from jax import lax
from jax.experimental import pallas as pl
from jax.experimental.pallas import tpu as pltpu
from jax.experimental.pallas import tpu_sc as plsc

N = 10000
E = 320000
G = 64
NC, NS = 2, 16
NPAD = 10240
RPT = NPAD // NS
CHUNK = 125
ECH = E // CHUNK
NCH = ECH // NS
DEPTH = 4
NB = 10
BLK = N // NB

_MESH = plsc.VectorSubcoreMesh(
    core_axis_name="c", subcore_axis_name="s", num_cores=NC, num_subcores=NS)


@functools.partial(
    pl.kernel,
    out_type=(jax.ShapeDtypeStruct((NC, NPAD), jnp.float32),
              jax.ShapeDtypeStruct((NC, NPAD), jnp.float32)),
    mesh=_MESH,
    scratch_types=[
        pltpu.VMEM_SHARED((NPAD,), jnp.float32),
        pltpu.VMEM_SHARED((NPAD,), jnp.float32),
        pltpu.VMEM((NCH, CHUNK), jnp.int32),
        pltpu.VMEM((NCH, CHUNK), jnp.int32),
        pltpu.VMEM((CHUNK,), jnp.float32),
        pltpu.SemaphoreType.DMA,
    ],
)
def _degree_hist(src_hbm, dst_hbm, zero_hbm, one_hbm, dego_hbm, degi_hbm,
                 ho_sp, hi_sp, idx_v, jdx_v, ones_v, sem):
  c = lax.axis_index("c")
  t = lax.axis_index("s")
  r0 = t * RPT
  pltpu.sync_copy(zero_hbm, ho_sp.at[pl.ds(r0, RPT)])
  pltpu.sync_copy(zero_hbm, hi_sp.at[pl.ds(r0, RPT)])
  pltpu.sync_copy(one_hbm, ones_v)
  pltpu.sync_copy(src_hbm.at[c].at[pl.ds(t * NCH, NCH)], idx_v)
  pltpu.sync_copy(dst_hbm.at[c].at[pl.ds(t * NCH, NCH)], jdx_v)
  plsc.subcore_barrier()

  K = 16

  def rounds(idx, sp):
    def body(i, _):
      for b in range(K):
        pltpu.async_copy(ones_v, sp.at[idx.at[i * K + b]], sem, add=True)
      for b in range(K):
        pltpu.make_async_copy(ones_v, sp.at[idx.at[i * K + b]], sem).wait()
      return 0
    lax.fori_loop(0, NCH // K, body, 0)

  rounds(idx_v, ho_sp)
  rounds(jdx_v, hi_sp)
  plsc.subcore_barrier()
  pltpu.sync_copy(ho_sp.at[pl.ds(r0, RPT)], dego_hbm.at[c].at[pl.ds(r0, RPT)])
  pltpu.sync_copy(hi_sp.at[pl.ds(r0, RPT)], degi_hbm.at[c].at[pl.ds(r0, RPT)])


def _make_edge_agg(F):
  @functools.partial(
      pl.kernel,
      out_type=jax.ShapeDtypeStruct((NC, NPAD, F), jnp.float32),
      mesh=_MESH,
      scratch_types=[
          pltpu.VMEM_SHARED((NPAD, F), jnp.float32),
          pltpu.VMEM((NCH, CHUNK), jnp.int32),
          pltpu.VMEM((NCH, CHUNK), jnp.int32),
          pltpu.VMEM((DEPTH, CHUNK, F), jnp.float32),
          pltpu.SemaphoreType.DMA,
      ],
      compiler_params=pltpu.CompilerParams(use_tc_tiling_on_sc=False),
  )
  def agg(tbl_hbm, src_hbm, dst_hbm, zero_hbm, out_hbm,
          acc_sp, isrc_v, idst_v, rows_v, gsem):
    c = lax.axis_index("c")
    t = lax.axis_index("s")
    r0 = t * RPT
    pltpu.sync_copy(zero_hbm, acc_sp.at[pl.ds(r0, RPT)])
    pltpu.sync_copy(src_hbm.at[c].at[pl.ds(t * NCH, NCH)], isrc_v)
    pltpu.sync_copy(dst_hbm.at[c].at[pl.ds(t * NCH, NCH)], idst_v)
    plsc.subcore_barrier()

    tbl = tbl_hbm.at[c]

    def fire(j, s):
      pltpu.async_copy(tbl.at[isrc_v.at[j]], rows_v.at[s], gsem)

    def drain(j, s):
      pltpu.make_async_copy(tbl.at[isrc_v.at[j]], rows_v.at[s], gsem).wait()

    for s in range(DEPTH):
      fire(s, s)

    def body(i, _):
      j0 = i * DEPTH
      for s in range(DEPTH):
        j = j0 + s
        drain(j, s)
        pltpu.sync_copy(rows_v.at[s], acc_sp.at[idst_v.at[j]], add=True)
        nj = j + DEPTH

        @pl.when(nj < NCH)
        def _():
          fire(nj, s)
      return 0

    lax.fori_loop(0, NCH // DEPTH, body, 0)
    plsc.subcore_barrier()
    pltpu.sync_copy(acc_sp.at[pl.ds(r0, RPT)], out_hbm.at[c].at[pl.ds(r0, RPT)])

  return agg


_edge_agg32 = _make_edge_agg(32)
_edge_agg16 = _make_edge_agg(16)


def _proj_scale(x, deg, W):
  din, dout = W.shape

  def body(x_ref, d_ref, w_ref, o_ref):
    a = lax.rsqrt(jnp.maximum(d_ref[...], 1.0))
    o_ref[...] = jnp.dot(x_ref[...], w_ref[...],
                         preferred_element_type=jnp.float32) * a

  return pl.pallas_call(
      body,
      grid=(2 * NB,),
      in_specs=[pl.BlockSpec((BLK, din), lambda i: (i, 0)),
                pl.BlockSpec((BLK, 1), lambda i: (i, 0)),
                pl.BlockSpec((din, dout), lambda i: (0, 0))],
      out_specs=pl.BlockSpec((BLK, dout), lambda i: (i, 0)),
      out_shape=jax.ShapeDtypeStruct((2 * N, dout), jnp.float32),
  )(x, deg, W)


def _post_proj_scale(agg, degi, dego, b, W):
  din, dout = W.shape

  def body(a_ref, di_ref, do_ref, b_ref, w_ref, o_ref):
    bi = lax.rsqrt(jnp.maximum(di_ref[...], 1.0))
    ao = lax.rsqrt(jnp.maximum(do_ref[...], 1.0))
    h = jnp.maximum(a_ref[...] * bi + b_ref[...], 0.0)
    o_ref[...] = jnp.dot(h, w_ref[...],
                         preferred_element_type=jnp.float32) * ao

  return pl.pallas_call(
      body,
      grid=(2 * NB,),
      in_specs=[pl.BlockSpec((BLK, din), lambda i: (i, 0)),
                pl.BlockSpec((BLK, 1), lambda i: (i, 0)),
                pl.BlockSpec((BLK, 1), lambda i: (i, 0)),
                pl.BlockSpec((1, din), lambda i: (0, 0)),
                pl.BlockSpec((din, dout), lambda i: (0, 0))],
      out_specs=pl.BlockSpec((BLK, dout), lambda i: (i, 0)),
      out_shape=jax.ShapeDtypeStruct((2 * N, dout), jnp.float32),
  )(agg, degi, dego, b, W)


def _readout_head(agg2, degi, b2, n2g, Wf, bf):
  def body(a_ref, di_ref, b_ref, g_ref, wf_ref, bf_ref, o_ref, r_acc, c_acc):
    i = pl.program_id(0)

    @pl.when(i == 0)
    def _():
      r_acc[...] = jnp.zeros_like(r_acc)
      c_acc[...] = jnp.zeros_like(c_acc)

    bi = lax.rsqrt(jnp.maximum(di_ref[...], 1.0))
    h = jnp.maximum(a_ref[...] * bi + b_ref[...], 0.0)
    gid = g_ref[...] + (i // NB) * G
    onehot = (gid == lax.broadcasted_iota(jnp.int32, (BLK, 2 * G), 1)
              ).astype(jnp.float32)
    r_acc[...] += lax.dot_general(onehot, h, (((0,), (0,)), ((), ())),
                                  preferred_element_type=jnp.float32)
    c_acc[...] += jnp.sum(onehot, axis=0)[:, None]

    @pl.when(i == 2 * NB - 1)
    def _():
      m = r_acc[...] / jnp.maximum(c_acc[...], 1.0)
      hidden = m[:G] + m[G:]
      o_ref[...] = jax.nn.sigmoid(
          jnp.dot(hidden, wf_ref[...], preferred_element_type=jnp.float32)
          + bf_ref[...])

  return pl.pallas_call(
      body,
      grid=(2 * NB,),
      in_specs=[pl.BlockSpec((BLK, 16), lambda i: (i, 0)),
                pl.BlockSpec((BLK, 1), lambda i: (i, 0)),
                pl.BlockSpec((1, 16), lambda i: (0, 0)),
                pl.BlockSpec((BLK, 1), lambda i: (i, 0)),
                pl.BlockSpec((16, 1), lambda i: (0, 0)),
                pl.BlockSpec((1, 1), lambda i: (0, 0))],
      out_specs=pl.BlockSpec((G, 1), lambda i: (0, 0)),
      out_shape=jax.ShapeDtypeStruct((G, 1), jnp.float32),
      scratch_shapes=[pltpu.VMEM((2 * G, 16), jnp.float32),
                      pltpu.VMEM((2 * G, 1), jnp.float32)],
  )(agg2, degi, b2, n2g, Wf, bf)


def kernel(x_left, x_right, edge_index_left, edge_index_right,
           node2graph_left, node2graph_right, W1, b1, W2, b2, Wf, bf):
  f32 = jnp.float32
  src2d = jnp.stack([edge_index_left[0].astype(jnp.int32).reshape(ECH, CHUNK),
                     edge_index_right[0].astype(jnp.int32).reshape(ECH, CHUNK)])
  dst2d = jnp.stack([edge_index_left[1].astype(jnp.int32).reshape(ECH, CHUNK),
                     edge_index_right[1].astype(jnp.int32).reshape(ECH, CHUNK)])
  zeros1 = jnp.zeros((RPT,), f32)
  ones1 = jnp.ones((CHUNK,), f32)
  zeros32 = jnp.zeros((RPT, 32), f32)
  zeros16 = jnp.zeros((RPT, 16), f32)

  dego, degi = _degree_hist(src2d, dst2d, zeros1, ones1)
  dego_cat = jnp.concatenate([dego[0, :N], dego[1, :N]]).reshape(2 * N, 1)
  degi_cat = jnp.concatenate([degi[0, :N], degi[1, :N]]).reshape(2 * N, 1)

  x_cat = jnp.concatenate([x_left, x_right], axis=0)
  y1 = _proj_scale(x_cat, dego_cat, W1)
  y1_tbl = jnp.stack([y1[:N], y1[N:]])
  agg1 = _edge_agg32(y1_tbl, src2d, dst2d, zeros32)
  agg1_cat = jnp.concatenate([agg1[0, :N], agg1[1, :N]], axis=0)

  y2 = _post_proj_scale(agg1_cat, degi_cat, dego_cat,
                        b1.reshape(1, 32), W2)
  y2_tbl = jnp.stack([y2[:N], y2[N:]])
  agg2 = _edge_agg16(y2_tbl, src2d, dst2d, zeros16)
  agg2_cat = jnp.concatenate([agg2[0, :N], agg2[1, :N]], axis=0)

  n2g_cat = jnp.concatenate([node2graph_left, node2graph_right]
                            ).astype(jnp.int32).reshape(2 * N, 1)
  return _readout_head(agg2_cat, degi_cat, b2.reshape(1, 16), n2g_cat,
                       Wf, bf.reshape(1, 1))


# speedup vs baseline: 42.2859x; 1.1807x over previous
import functools

import jax
import jax.numpy as jnp
from jax import lax
from jax.experimental import pallas as pl
from jax.experimental.pallas import tpu as pltpu
from jax.experimental.pallas import tpu_sc as plsc

N = 10000
E = 320000
G = 64
NC, NS = 2, 16
OWN = 640
OWN_LAST = N - OWN * (NS - 1)
CHUNK = 125
ECH = E // CHUNK
NCH = ECH // NS
DEPTH = 4
NB = 10
BLK = N // NB

_MESH = plsc.VectorSubcoreMesh(
    core_axis_name="c", subcore_axis_name="s", num_cores=NC, num_subcores=NS)


def _per_tile_rows(t, fn):
  @pl.when(t < NS - 1)
  def _():
    fn(t * OWN, OWN)

  @pl.when(t == NS - 1)
  def _():
    fn((NS - 1) * OWN, OWN_LAST)


@functools.partial(
    pl.kernel,
    out_type=(jax.ShapeDtypeStruct((NC, N), jnp.float32),
              jax.ShapeDtypeStruct((NC, N), jnp.float32)),
    mesh=_MESH,
    scratch_types=[
        pltpu.VMEM_SHARED((N,), jnp.float32),
        pltpu.VMEM_SHARED((N,), jnp.float32),
        pltpu.VMEM((NCH, CHUNK), jnp.int32),
        pltpu.VMEM((NCH, CHUNK), jnp.int32),
        pltpu.VMEM((CHUNK,), jnp.float32),
        pltpu.SemaphoreType.DMA,
    ],
    compiler_params=pltpu.CompilerParams(use_tc_tiling_on_sc=False),
)
def _degree_hist(srcl, dstl, srcr, dstr, zero_hbm, one_hbm, dego_hbm, degi_hbm,
                 ho_sp, hi_sp, idx_v, jdx_v, ones_v, sem):
  c = lax.axis_index("c")
  t = lax.axis_index("s")

  def zero(r0, sz):
    pltpu.sync_copy(zero_hbm.at[pl.ds(0, sz)], ho_sp.at[pl.ds(r0, sz)])
    pltpu.sync_copy(zero_hbm.at[pl.ds(0, sz)], hi_sp.at[pl.ds(r0, sz)])

  _per_tile_rows(t, zero)
  pltpu.sync_copy(one_hbm, ones_v)

  @pl.when(c == 0)
  def _():
    pltpu.sync_copy(srcl.at[pl.ds(t * NCH, NCH)], idx_v)
    pltpu.sync_copy(dstl.at[pl.ds(t * NCH, NCH)], jdx_v)

  @pl.when(c == 1)
  def _():
    pltpu.sync_copy(srcr.at[pl.ds(t * NCH, NCH)], idx_v)
    pltpu.sync_copy(dstr.at[pl.ds(t * NCH, NCH)], jdx_v)

  plsc.subcore_barrier()

  K = 16

  def rounds(idx, sp):
    def body(i, _):
      for b in range(K):
        pltpu.async_copy(ones_v, sp.at[idx.at[i * K + b]], sem, add=True)
      for b in range(K):
        pltpu.make_async_copy(ones_v, sp.at[idx.at[i * K + b]], sem).wait()
      return 0
    lax.fori_loop(0, NCH // K, body, 0)

  rounds(idx_v, ho_sp)
  rounds(jdx_v, hi_sp)
  plsc.subcore_barrier()

  def flush(r0, sz):
    pltpu.sync_copy(ho_sp.at[pl.ds(r0, sz)], dego_hbm.at[c].at[pl.ds(r0, sz)])
    pltpu.sync_copy(hi_sp.at[pl.ds(r0, sz)], degi_hbm.at[c].at[pl.ds(r0, sz)])

  _per_tile_rows(t, flush)


def _make_edge_agg(F):
  @functools.partial(
      pl.kernel,
      out_type=jax.ShapeDtypeStruct((NC, N, F), jnp.float32),
      mesh=_MESH,
      scratch_types=[
          pltpu.VMEM_SHARED((N, F), jnp.float32),
          pltpu.VMEM((NCH, CHUNK), jnp.int32),
          pltpu.VMEM((NCH, CHUNK), jnp.int32),
          pltpu.VMEM((DEPTH, CHUNK, F), jnp.float32),
          pltpu.SemaphoreType.DMA,
      ],
      compiler_params=pltpu.CompilerParams(use_tc_tiling_on_sc=False),
  )
  def agg(tbl_hbm, srcl, dstl, srcr, dstr, zero_hbm, out_hbm,
          acc_sp, isrc_v, idst_v, rows_v, gsem):
    c = lax.axis_index("c")
    t = lax.axis_index("s")

    def zero(r0, sz):
      pltpu.sync_copy(zero_hbm.at[pl.ds(0, sz)], acc_sp.at[pl.ds(r0, sz)])

    _per_tile_rows(t, zero)

    @pl.when(c == 0)
    def _():
      pltpu.sync_copy(srcl.at[pl.ds(t * NCH, NCH)], isrc_v)
      pltpu.sync_copy(dstl.at[pl.ds(t * NCH, NCH)], idst_v)

    @pl.when(c == 1)
    def _():
      pltpu.sync_copy(srcr.at[pl.ds(t * NCH, NCH)], isrc_v)
      pltpu.sync_copy(dstr.at[pl.ds(t * NCH, NCH)], idst_v)

    plsc.subcore_barrier()

    tbl = tbl_hbm.at[c]

    def fire(j, s):
      pltpu.async_copy(tbl.at[isrc_v.at[j]], rows_v.at[s], gsem)

    def drain(j, s):
      pltpu.make_async_copy(tbl.at[isrc_v.at[j]], rows_v.at[s], gsem).wait()

    for s in range(DEPTH):
      fire(s, s)

    def body(i, _):
      j0 = i * DEPTH
      for s in range(DEPTH):
        j = j0 + s
        drain(j, s)
        pltpu.sync_copy(rows_v.at[s], acc_sp.at[idst_v.at[j]], add=True)
        nj = j + DEPTH

        @pl.when(nj < NCH)
        def _():
          fire(nj, s)
      return 0

    lax.fori_loop(0, NCH // DEPTH, body, 0)
    plsc.subcore_barrier()

    def flush(r0, sz):
      pltpu.sync_copy(acc_sp.at[pl.ds(r0, sz)],
                      out_hbm.at[c].at[pl.ds(r0, sz)])

    _per_tile_rows(t, flush)

  return agg


_edge_agg32 = _make_edge_agg(32)
_edge_agg16 = _make_edge_agg(16)


def _proj_scale(x, deg, W):
  din, dout = W.shape

  def body(x_ref, d_ref, w_ref, o_ref):
    a = lax.rsqrt(jnp.maximum(d_ref[...], 1.0))
    o_ref[...] = jnp.dot(x_ref[...], w_ref[...],
                         preferred_element_type=jnp.float32) * a

  return pl.pallas_call(
      body,
      grid=(2 * NB,),
      in_specs=[pl.BlockSpec((BLK, din), lambda i: (i, 0)),
                pl.BlockSpec((BLK, 1), lambda i: (i, 0)),
                pl.BlockSpec((din, dout), lambda i: (0, 0))],
      out_specs=pl.BlockSpec((BLK, dout), lambda i: (i, 0)),
      out_shape=jax.ShapeDtypeStruct((2 * N, dout), jnp.float32),
  )(x, deg, W)


def _post_proj_scale(agg, degi, dego, b, W):
  din, dout = W.shape

  def body(a_ref, di_ref, do_ref, b_ref, w_ref, o_ref):
    bi = lax.rsqrt(jnp.maximum(di_ref[...], 1.0))
    ao = lax.rsqrt(jnp.maximum(do_ref[...], 1.0))
    h = jnp.maximum(a_ref[...] * bi + b_ref[...], 0.0)
    o_ref[...] = jnp.dot(h, w_ref[...],
                         preferred_element_type=jnp.float32) * ao

  return pl.pallas_call(
      body,
      grid=(2 * NB,),
      in_specs=[pl.BlockSpec((BLK, din), lambda i: (i, 0)),
                pl.BlockSpec((BLK, 1), lambda i: (i, 0)),
                pl.BlockSpec((BLK, 1), lambda i: (i, 0)),
                pl.BlockSpec((1, din), lambda i: (0, 0)),
                pl.BlockSpec((din, dout), lambda i: (0, 0))],
      out_specs=pl.BlockSpec((BLK, dout), lambda i: (i, 0)),
      out_shape=jax.ShapeDtypeStruct((2 * N, dout), jnp.float32),
  )(agg, degi, dego, b, W)


def _readout_head(agg2, degi, b2, n2g, Wf, bf):
  def body(a_ref, di_ref, b_ref, g_ref, wf_ref, bf_ref, o_ref, r_acc, c_acc):
    i = pl.program_id(0)

    @pl.when(i == 0)
    def _():
      r_acc[...] = jnp.zeros_like(r_acc)
      c_acc[...] = jnp.zeros_like(c_acc)

    bi = lax.rsqrt(jnp.maximum(di_ref[...], 1.0))
    h = jnp.maximum(a_ref[...] * bi + b_ref[...], 0.0)
    gid = g_ref[...] + (i // NB) * G
    onehot = (gid == lax.broadcasted_iota(jnp.int32, (BLK, 2 * G), 1)
              ).astype(jnp.float32)
    r_acc[...] += lax.dot_general(onehot, h, (((0,), (0,)), ((), ())),
                                  preferred_element_type=jnp.float32)
    c_acc[...] += jnp.sum(onehot, axis=0)[:, None]

    @pl.when(i == 2 * NB - 1)
    def _():
      m = r_acc[...] / jnp.maximum(c_acc[...], 1.0)
      hidden = m[:G] + m[G:]
      o_ref[...] = jax.nn.sigmoid(
          jnp.dot(hidden, wf_ref[...], preferred_element_type=jnp.float32)
          + bf_ref[...])

  return pl.pallas_call(
      body,
      grid=(2 * NB,),
      in_specs=[pl.BlockSpec((BLK, 16), lambda i: (i, 0)),
                pl.BlockSpec((BLK, 1), lambda i: (i, 0)),
                pl.BlockSpec((1, 16), lambda i: (0, 0)),
                pl.BlockSpec((BLK, 1), lambda i: (i, 0)),
                pl.BlockSpec((16, 1), lambda i: (0, 0)),
                pl.BlockSpec((1, 1), lambda i: (0, 0))],
      out_specs=pl.BlockSpec((G, 1), lambda i: (0, 0)),
      out_shape=jax.ShapeDtypeStruct((G, 1), jnp.float32),
      scratch_shapes=[pltpu.VMEM((2 * G, 16), jnp.float32),
                      pltpu.VMEM((2 * G, 1), jnp.float32)],
  )(agg2, degi, b2, n2g, Wf, bf)


def kernel(x_left, x_right, edge_index_left, edge_index_right,
           node2graph_left, node2graph_right, W1, b1, W2, b2, Wf, bf):
  f32 = jnp.float32
  srcl = edge_index_left[0].astype(jnp.int32).reshape(ECH, CHUNK)
  dstl = edge_index_left[1].astype(jnp.int32).reshape(ECH, CHUNK)
  srcr = edge_index_right[0].astype(jnp.int32).reshape(ECH, CHUNK)
  dstr = edge_index_right[1].astype(jnp.int32).reshape(ECH, CHUNK)
  zeros1 = jnp.zeros((OWN,), f32)
  ones1 = jnp.ones((CHUNK,), f32)
  zeros32 = jnp.zeros((OWN, 32), f32)
  zeros16 = jnp.zeros((OWN, 16), f32)

  dego, degi = _degree_hist(srcl, dstl, srcr, dstr, zeros1, ones1)
  dego_cat = dego.reshape(2 * N, 1)
  degi_cat = degi.reshape(2 * N, 1)

  x_cat = jnp.concatenate([x_left, x_right], axis=0)
  y1 = _proj_scale(x_cat, dego_cat, W1)
  agg1 = _edge_agg32(y1.reshape(NC, N, 32), srcl, dstl, srcr, dstr,
                     zeros32)

  y2 = _post_proj_scale(agg1.reshape(2 * N, 32), degi_cat, dego_cat,
                        b1.reshape(1, 32), W2)
  agg2 = _edge_agg16(y2.reshape(NC, N, 16), srcl, dstl, srcr, dstr,
                     zeros16)

  n2g_cat = jnp.concatenate([node2graph_left, node2graph_right]
                            ).astype(jnp.int32).reshape(2 * N, 1)
  return _readout_head(agg2.reshape(2 * N, 16), degi_cat,
                       b2.reshape(1, 16), n2g_cat, Wf, bf.reshape(1, 1))


# speedup vs baseline: 45.2713x; 1.0706x over previous
import functools

import jax
import jax.numpy as jnp
from jax import lax
from jax.experimental import pallas as pl
from jax.experimental.pallas import tpu as pltpu
from jax.experimental.pallas import tpu_sc as plsc

N = 10000
E = 320000
G = 64
NC, NS = 2, 16
OWN = 640
OWN_LAST = N - OWN * (NS - 1)
CHUNK = 125
ECH = E // CHUNK
NCH = ECH // NS
NBUF = 8
GAH = 4
NB = 5
BLK = N // NB

_MESH = plsc.VectorSubcoreMesh(
    core_axis_name="c", subcore_axis_name="s", num_cores=NC, num_subcores=NS)


def _per_tile_rows(t, fn):
  @pl.when(t < NS - 1)
  def _():
    fn(t * OWN, OWN)

  @pl.when(t == NS - 1)
  def _():
    fn((NS - 1) * OWN, OWN_LAST)


@functools.partial(
    pl.kernel,
    out_type=(jax.ShapeDtypeStruct((NC, N), jnp.float32),
              jax.ShapeDtypeStruct((NC, N), jnp.float32)),
    mesh=_MESH,
    scratch_types=[
        pltpu.VMEM_SHARED((N,), jnp.float32),
        pltpu.VMEM_SHARED((N,), jnp.float32),
        pltpu.VMEM((NCH, CHUNK), jnp.int32),
        pltpu.VMEM((NCH, CHUNK), jnp.int32),
        pltpu.VMEM((CHUNK,), jnp.float32),
        pltpu.SemaphoreType.DMA,
    ],
    compiler_params=pltpu.CompilerParams(use_tc_tiling_on_sc=False),
)
def _degree_hist(srcl, dstl, srcr, dstr, zero_hbm, one_hbm, dego_hbm, degi_hbm,
                 ho_sp, hi_sp, idx_v, jdx_v, ones_v, sem):
  c = lax.axis_index("c")
  t = lax.axis_index("s")

  def zero(r0, sz):
    pltpu.sync_copy(zero_hbm.at[pl.ds(0, sz)], ho_sp.at[pl.ds(r0, sz)])
    pltpu.sync_copy(zero_hbm.at[pl.ds(0, sz)], hi_sp.at[pl.ds(r0, sz)])

  _per_tile_rows(t, zero)
  pltpu.sync_copy(one_hbm, ones_v)

  @pl.when(c == 0)
  def _():
    pltpu.sync_copy(srcl.at[pl.ds(t * NCH, NCH)], idx_v)
    pltpu.sync_copy(dstl.at[pl.ds(t * NCH, NCH)], jdx_v)

  @pl.when(c == 1)
  def _():
    pltpu.sync_copy(srcr.at[pl.ds(t * NCH, NCH)], idx_v)
    pltpu.sync_copy(dstr.at[pl.ds(t * NCH, NCH)], jdx_v)

  plsc.subcore_barrier()

  K = 16

  def rounds(idx, sp):
    def body(i, _):
      for b in range(K):
        pltpu.async_copy(ones_v, sp.at[idx.at[i * K + b]], sem, add=True)
      for b in range(K):
        pltpu.make_async_copy(ones_v, sp.at[idx.at[i * K + b]], sem).wait()
      return 0
    lax.fori_loop(0, NCH // K, body, 0)

  rounds(idx_v, ho_sp)
  rounds(jdx_v, hi_sp)
  plsc.subcore_barrier()

  def flush(r0, sz):
    pltpu.sync_copy(ho_sp.at[pl.ds(r0, sz)], dego_hbm.at[c].at[pl.ds(r0, sz)])
    pltpu.sync_copy(hi_sp.at[pl.ds(r0, sz)], degi_hbm.at[c].at[pl.ds(r0, sz)])

  _per_tile_rows(t, flush)


def _make_edge_agg(F):
  @functools.partial(
      pl.kernel,
      out_type=jax.ShapeDtypeStruct((NC, N, F), jnp.float32),
      mesh=_MESH,
      scratch_types=[
          pltpu.VMEM_SHARED((N, F), jnp.float32),
          pltpu.VMEM((NCH, CHUNK), jnp.int32),
          pltpu.VMEM((NCH, CHUNK), jnp.int32),
          pltpu.VMEM((NBUF, CHUNK, F), jnp.float32),
          pltpu.SemaphoreType.DMA,
          pltpu.SemaphoreType.DMA((NBUF,)),
      ],
      compiler_params=pltpu.CompilerParams(use_tc_tiling_on_sc=False),
  )
  def agg(tbl_hbm, srcl, dstl, srcr, dstr, zero_hbm, out_hbm,
          acc_sp, isrc_v, idst_v, rows_v, gsem, ssem):
    c = lax.axis_index("c")
    t = lax.axis_index("s")

    def zero(r0, sz):
      pltpu.sync_copy(zero_hbm.at[pl.ds(0, sz)], acc_sp.at[pl.ds(r0, sz)])

    _per_tile_rows(t, zero)

    @pl.when(c == 0)
    def _():
      pltpu.sync_copy(srcl.at[pl.ds(t * NCH, NCH)], isrc_v)
      pltpu.sync_copy(dstl.at[pl.ds(t * NCH, NCH)], idst_v)

    @pl.when(c == 1)
    def _():
      pltpu.sync_copy(srcr.at[pl.ds(t * NCH, NCH)], isrc_v)
      pltpu.sync_copy(dstr.at[pl.ds(t * NCH, NCH)], idst_v)

    plsc.subcore_barrier()

    tbl = tbl_hbm.at[c]

    def fire_gather(j, s):
      pltpu.async_copy(tbl.at[isrc_v.at[j]], rows_v.at[s], gsem)

    def wait_gather(j, s):
      pltpu.make_async_copy(tbl.at[isrc_v.at[j]], rows_v.at[s], gsem).wait()

    def fire_scatter(j, s):
      pltpu.async_copy(rows_v.at[s], acc_sp.at[idst_v.at[j]], ssem.at[s],
                       add=True)

    def wait_scatter(j, s):
      pltpu.make_async_copy(rows_v.at[s], acc_sp.at[idst_v.at[j]],
                            ssem.at[s]).wait()

    for s in range(GAH):
      fire_gather(s, s)

    def body(i, _):
      j0 = i * NBUF
      for s in range(NBUF):
        j = j0 + s
        wait_gather(j, s)
        fire_scatter(j, s)
        f = j + GAH
        sf = (s + GAH) % NBUF

        @pl.when(f < NCH)
        def _():
          @pl.when(j >= GAH)
          def _():
            wait_scatter(j - GAH, sf)
          fire_gather(f, sf)
      return 0

    lax.fori_loop(0, NCH // NBUF, body, 0)
    for s in range(NBUF):
      wait_scatter(NCH - NBUF + s, s)
    plsc.subcore_barrier()

    def flush(r0, sz):
      pltpu.sync_copy(acc_sp.at[pl.ds(r0, sz)],
                      out_hbm.at[c].at[pl.ds(r0, sz)])

    _per_tile_rows(t, flush)

  return agg


_edge_agg32 = _make_edge_agg(32)
_edge_agg16 = _make_edge_agg(16)


def _proj_scale(x, deg, W):
  din, dout = W.shape

  def body(x_ref, d_ref, w_ref, o_ref):
    a = lax.rsqrt(jnp.maximum(d_ref[...], 1.0))
    o_ref[...] = jnp.dot(x_ref[...], w_ref[...],
                         preferred_element_type=jnp.float32) * a

  return pl.pallas_call(
      body,
      grid=(2 * NB,),
      in_specs=[pl.BlockSpec((BLK, din), lambda i: (i, 0)),
                pl.BlockSpec((BLK, 1), lambda i: (i, 0)),
                pl.BlockSpec((din, dout), lambda i: (0, 0))],
      out_specs=pl.BlockSpec((BLK, dout), lambda i: (i, 0)),
      out_shape=jax.ShapeDtypeStruct((2 * N, dout), jnp.float32),
  )(x, deg, W)


def _post_proj_scale(agg, degi, dego, b, W):
  din, dout = W.shape

  def body(a_ref, di_ref, do_ref, b_ref, w_ref, o_ref):
    bi = lax.rsqrt(jnp.maximum(di_ref[...], 1.0))
    ao = lax.rsqrt(jnp.maximum(do_ref[...], 1.0))
    h = jnp.maximum(a_ref[...] * bi + b_ref[...], 0.0)
    o_ref[...] = jnp.dot(h, w_ref[...],
                         preferred_element_type=jnp.float32) * ao

  return pl.pallas_call(
      body,
      grid=(2 * NB,),
      in_specs=[pl.BlockSpec((BLK, din), lambda i: (i, 0)),
                pl.BlockSpec((BLK, 1), lambda i: (i, 0)),
                pl.BlockSpec((BLK, 1), lambda i: (i, 0)),
                pl.BlockSpec((1, din), lambda i: (0, 0)),
                pl.BlockSpec((din, dout), lambda i: (0, 0))],
      out_specs=pl.BlockSpec((BLK, dout), lambda i: (i, 0)),
      out_shape=jax.ShapeDtypeStruct((2 * N, dout), jnp.float32),
  )(agg, degi, dego, b, W)


def _readout_head(agg2, degi, b2, n2g, Wf, bf):
  def body(a_ref, di_ref, b_ref, g_ref, wf_ref, bf_ref, o_ref, r_acc, c_acc):
    i = pl.program_id(0)

    @pl.when(i == 0)
    def _():
      r_acc[...] = jnp.zeros_like(r_acc)
      c_acc[...] = jnp.zeros_like(c_acc)

    bi = lax.rsqrt(jnp.maximum(di_ref[...], 1.0))
    h = jnp.maximum(a_ref[...] * bi + b_ref[...], 0.0)
    gid = g_ref[...] + (i // NB) * G
    onehot = (gid == lax.broadcasted_iota(jnp.int32, (BLK, 2 * G), 1)
              ).astype(jnp.float32)
    r_acc[...] += lax.dot_general(onehot, h, (((0,), (0,)), ((), ())),
                                  preferred_element_type=jnp.float32)
    c_acc[...] += jnp.sum(onehot, axis=0)[:, None]

    @pl.when(i == 2 * NB - 1)
    def _():
      m = r_acc[...] / jnp.maximum(c_acc[...], 1.0)
      hidden = m[:G] + m[G:]
      o_ref[...] = jax.nn.sigmoid(
          jnp.dot(hidden, wf_ref[...], preferred_element_type=jnp.float32)
          + bf_ref[...])

  return pl.pallas_call(
      body,
      grid=(2 * NB,),
      in_specs=[pl.BlockSpec((BLK, 16), lambda i: (i, 0)),
                pl.BlockSpec((BLK, 1), lambda i: (i, 0)),
                pl.BlockSpec((1, 16), lambda i: (0, 0)),
                pl.BlockSpec((BLK, 1), lambda i: (i, 0)),
                pl.BlockSpec((16, 1), lambda i: (0, 0)),
                pl.BlockSpec((1, 1), lambda i: (0, 0))],
      out_specs=pl.BlockSpec((G, 1), lambda i: (0, 0)),
      out_shape=jax.ShapeDtypeStruct((G, 1), jnp.float32),
      scratch_shapes=[pltpu.VMEM((2 * G, 16), jnp.float32),
                      pltpu.VMEM((2 * G, 1), jnp.float32)],
  )(agg2, degi, b2, n2g, Wf, bf)


def kernel(x_left, x_right, edge_index_left, edge_index_right,
           node2graph_left, node2graph_right, W1, b1, W2, b2, Wf, bf):
  f32 = jnp.float32
  srcl = edge_index_left[0].astype(jnp.int32).reshape(ECH, CHUNK)
  dstl = edge_index_left[1].astype(jnp.int32).reshape(ECH, CHUNK)
  srcr = edge_index_right[0].astype(jnp.int32).reshape(ECH, CHUNK)
  dstr = edge_index_right[1].astype(jnp.int32).reshape(ECH, CHUNK)
  zeros1 = jnp.zeros((OWN,), f32)
  ones1 = jnp.ones((CHUNK,), f32)
  zeros32 = jnp.zeros((OWN, 32), f32)
  zeros16 = jnp.zeros((OWN, 16), f32)

  dego, degi = _degree_hist(srcl, dstl, srcr, dstr, zeros1, ones1)
  dego_cat = dego.reshape(2 * N, 1)
  degi_cat = degi.reshape(2 * N, 1)

  x_cat = jnp.concatenate([x_left, x_right], axis=0)
  y1 = _proj_scale(x_cat, dego_cat, W1)
  agg1 = _edge_agg32(y1.reshape(NC, N, 32), srcl, dstl, srcr, dstr,
                     zeros32)

  y2 = _post_proj_scale(agg1.reshape(2 * N, 32), degi_cat, dego_cat,
                        b1.reshape(1, 32), W2)
  agg2 = _edge_agg16(y2.reshape(NC, N, 16), srcl, dstl, srcr, dstr,
                     zeros16)

  n2g_cat = jnp.concatenate([node2graph_left, node2graph_right]
                            ).astype(jnp.int32).reshape(2 * N, 1)
  return _readout_head(agg2.reshape(2 * N, 16), degi_cat,
                       b2.reshape(1, 16), n2g_cat, Wf, bf.reshape(1, 1))


# speedup vs baseline: 50.7180x; 1.1203x over previous
import functools

import jax
import jax.numpy as jnp
from jax import lax
from jax.experimental import pallas as pl
from jax.experimental.pallas import tpu as pltpu
from jax.experimental.pallas import tpu_sc as plsc

N = 10000
E = 320000
G = 64
NC, NS = 2, 16
OWN = 640
OWN_LAST = N - OWN * (NS - 1)
CHUNK = 125
ECH = E // CHUNK
NCH = ECH // NS
NBUF = 16
GAH = 8
NB = 2
BLK = N // NB

_MESH = plsc.VectorSubcoreMesh(
    core_axis_name="c", subcore_axis_name="s", num_cores=NC, num_subcores=NS)


def _per_tile_rows(t, fn):
  @pl.when(t < NS - 1)
  def _():
    fn(t * OWN, OWN)

  @pl.when(t == NS - 1)
  def _():
    fn((NS - 1) * OWN, OWN_LAST)


@functools.partial(
    pl.kernel,
    out_type=(jax.ShapeDtypeStruct((NC, N), jnp.float32),
              jax.ShapeDtypeStruct((NC, N), jnp.float32)),
    mesh=_MESH,
    scratch_types=[
        pltpu.VMEM_SHARED((N,), jnp.float32),
        pltpu.VMEM_SHARED((N,), jnp.float32),
        pltpu.VMEM((NCH, CHUNK), jnp.int32),
        pltpu.VMEM((NCH, CHUNK), jnp.int32),
        pltpu.VMEM((CHUNK,), jnp.float32),
        pltpu.SemaphoreType.DMA,
    ],
    compiler_params=pltpu.CompilerParams(use_tc_tiling_on_sc=False),
)
def _degree_hist(srcl, dstl, srcr, dstr, zero_hbm, one_hbm, dego_hbm, degi_hbm,
                 ho_sp, hi_sp, idx_v, jdx_v, ones_v, sem):
  c = lax.axis_index("c")
  t = lax.axis_index("s")

  def zero(r0, sz):
    pltpu.sync_copy(zero_hbm.at[pl.ds(0, sz)], ho_sp.at[pl.ds(r0, sz)])
    pltpu.sync_copy(zero_hbm.at[pl.ds(0, sz)], hi_sp.at[pl.ds(r0, sz)])

  _per_tile_rows(t, zero)
  pltpu.sync_copy(one_hbm, ones_v)

  @pl.when(c == 0)
  def _():
    pltpu.sync_copy(srcl.at[pl.ds(t * NCH, NCH)], idx_v)
    pltpu.sync_copy(dstl.at[pl.ds(t * NCH, NCH)], jdx_v)

  @pl.when(c == 1)
  def _():
    pltpu.sync_copy(srcr.at[pl.ds(t * NCH, NCH)], idx_v)
    pltpu.sync_copy(dstr.at[pl.ds(t * NCH, NCH)], jdx_v)

  plsc.subcore_barrier()

  K = 16

  def rounds(idx, sp):
    def body(i, _):
      for b in range(K):
        pltpu.async_copy(ones_v, sp.at[idx.at[i * K + b]], sem, add=True)
      for b in range(K):
        pltpu.make_async_copy(ones_v, sp.at[idx.at[i * K + b]], sem).wait()
      return 0
    lax.fori_loop(0, NCH // K, body, 0)

  rounds(idx_v, ho_sp)
  rounds(jdx_v, hi_sp)
  plsc.subcore_barrier()

  def flush(r0, sz):
    pltpu.sync_copy(ho_sp.at[pl.ds(r0, sz)], dego_hbm.at[c].at[pl.ds(r0, sz)])
    pltpu.sync_copy(hi_sp.at[pl.ds(r0, sz)], degi_hbm.at[c].at[pl.ds(r0, sz)])

  _per_tile_rows(t, flush)


def _make_edge_agg(F):
  @functools.partial(
      pl.kernel,
      out_type=jax.ShapeDtypeStruct((NC, N, F), jnp.float32),
      mesh=_MESH,
      scratch_types=[
          pltpu.VMEM_SHARED((N, F), jnp.float32),
          pltpu.VMEM((NCH, CHUNK), jnp.int32),
          pltpu.VMEM((NCH, CHUNK), jnp.int32),
          pltpu.VMEM((NBUF, CHUNK, F), jnp.float32),
          pltpu.SemaphoreType.DMA,
          pltpu.SemaphoreType.DMA((NBUF,)),
      ],
      compiler_params=pltpu.CompilerParams(use_tc_tiling_on_sc=False),
  )
  def agg(tbl_hbm, srcl, dstl, srcr, dstr, zero_hbm, out_hbm,
          acc_sp, isrc_v, idst_v, rows_v, gsem, ssem):
    c = lax.axis_index("c")
    t = lax.axis_index("s")

    def zero(r0, sz):
      pltpu.sync_copy(zero_hbm.at[pl.ds(0, sz)], acc_sp.at[pl.ds(r0, sz)])

    _per_tile_rows(t, zero)

    @pl.when(c == 0)
    def _():
      pltpu.sync_copy(srcl.at[pl.ds(t * NCH, NCH)], isrc_v)
      pltpu.sync_copy(dstl.at[pl.ds(t * NCH, NCH)], idst_v)

    @pl.when(c == 1)
    def _():
      pltpu.sync_copy(srcr.at[pl.ds(t * NCH, NCH)], isrc_v)
      pltpu.sync_copy(dstr.at[pl.ds(t * NCH, NCH)], idst_v)

    plsc.subcore_barrier()

    tbl = tbl_hbm.at[c]

    def fire_gather(j, s):
      pltpu.async_copy(tbl.at[isrc_v.at[j]], rows_v.at[s], gsem)

    def wait_gather(j, s):
      pltpu.make_async_copy(tbl.at[isrc_v.at[j]], rows_v.at[s], gsem).wait()

    def fire_scatter(j, s):
      pltpu.async_copy(rows_v.at[s], acc_sp.at[idst_v.at[j]], ssem.at[s],
                       add=True)

    def wait_scatter(j, s):
      pltpu.make_async_copy(rows_v.at[s], acc_sp.at[idst_v.at[j]],
                            ssem.at[s]).wait()

    for s in range(GAH):
      fire_gather(s, s)

    def body(i, _):
      j0 = i * NBUF
      for s in range(NBUF):
        j = j0 + s
        wait_gather(j, s)
        fire_scatter(j, s)
        f = j + GAH
        sf = (s + GAH) % NBUF

        @pl.when(f < NCH)
        def _():
          @pl.when(j >= GAH)
          def _():
            wait_scatter(j - GAH, sf)
          fire_gather(f, sf)
      return 0

    lax.fori_loop(0, NCH // NBUF, body, 0)
    for s in range(NBUF):
      wait_scatter(NCH - NBUF + s, s)
    plsc.subcore_barrier()

    def flush(r0, sz):
      pltpu.sync_copy(acc_sp.at[pl.ds(r0, sz)],
                      out_hbm.at[c].at[pl.ds(r0, sz)])

    _per_tile_rows(t, flush)

  return agg


_edge_agg32 = _make_edge_agg(32)
_edge_agg16 = _make_edge_agg(16)


def _proj_scale(x, deg, W):
  din, dout = W.shape

  def body(x_ref, d_ref, w_ref, o_ref):
    a = lax.rsqrt(jnp.maximum(d_ref[...], 1.0))
    o_ref[...] = jnp.dot(x_ref[...], w_ref[...],
                         preferred_element_type=jnp.float32) * a

  return pl.pallas_call(
      body,
      grid=(2 * NB,),
      in_specs=[pl.BlockSpec((BLK, din), lambda i: (i, 0)),
                pl.BlockSpec((BLK, 1), lambda i: (i, 0)),
                pl.BlockSpec((din, dout), lambda i: (0, 0))],
      out_specs=pl.BlockSpec((BLK, dout), lambda i: (i, 0)),
      out_shape=jax.ShapeDtypeStruct((2 * N, dout), jnp.float32),
  )(x, deg, W)


def _post_proj_scale(agg, degi, dego, b, W):
  din, dout = W.shape

  def body(a_ref, di_ref, do_ref, b_ref, w_ref, o_ref):
    bi = lax.rsqrt(jnp.maximum(di_ref[...], 1.0))
    ao = lax.rsqrt(jnp.maximum(do_ref[...], 1.0))
    h = jnp.maximum(a_ref[...] * bi + b_ref[...], 0.0)
    o_ref[...] = jnp.dot(h, w_ref[...],
                         preferred_element_type=jnp.float32) * ao

  return pl.pallas_call(
      body,
      grid=(2 * NB,),
      in_specs=[pl.BlockSpec((BLK, din), lambda i: (i, 0)),
                pl.BlockSpec((BLK, 1), lambda i: (i, 0)),
                pl.BlockSpec((BLK, 1), lambda i: (i, 0)),
                pl.BlockSpec((1, din), lambda i: (0, 0)),
                pl.BlockSpec((din, dout), lambda i: (0, 0))],
      out_specs=pl.BlockSpec((BLK, dout), lambda i: (i, 0)),
      out_shape=jax.ShapeDtypeStruct((2 * N, dout), jnp.float32),
  )(agg, degi, dego, b, W)


def _readout_head(agg2, degi, b2, n2g, Wf, bf):
  def body(a_ref, di_ref, b_ref, g_ref, wf_ref, bf_ref, o_ref, r_acc, c_acc):
    i = pl.program_id(0)

    @pl.when(i == 0)
    def _():
      r_acc[...] = jnp.zeros_like(r_acc)
      c_acc[...] = jnp.zeros_like(c_acc)

    bi = lax.rsqrt(jnp.maximum(di_ref[...], 1.0))
    h = jnp.maximum(a_ref[...] * bi + b_ref[...], 0.0)
    gid = g_ref[...] + (i // NB) * G
    onehot = (gid == lax.broadcasted_iota(jnp.int32, (BLK, 2 * G), 1)
              ).astype(jnp.float32)
    r_acc[...] += lax.dot_general(onehot, h, (((0,), (0,)), ((), ())),
                                  preferred_element_type=jnp.float32)
    c_acc[...] += jnp.sum(onehot, axis=0)[:, None]

    @pl.when(i == 2 * NB - 1)
    def _():
      m = r_acc[...] / jnp.maximum(c_acc[...], 1.0)
      hidden = m[:G] + m[G:]
      o_ref[...] = jax.nn.sigmoid(
          jnp.dot(hidden, wf_ref[...], preferred_element_type=jnp.float32)
          + bf_ref[...])

  return pl.pallas_call(
      body,
      grid=(2 * NB,),
      in_specs=[pl.BlockSpec((BLK, 16), lambda i: (i, 0)),
                pl.BlockSpec((BLK, 1), lambda i: (i, 0)),
                pl.BlockSpec((1, 16), lambda i: (0, 0)),
                pl.BlockSpec((BLK, 1), lambda i: (i, 0)),
                pl.BlockSpec((16, 1), lambda i: (0, 0)),
                pl.BlockSpec((1, 1), lambda i: (0, 0))],
      out_specs=pl.BlockSpec((G, 1), lambda i: (0, 0)),
      out_shape=jax.ShapeDtypeStruct((G, 1), jnp.float32),
      scratch_shapes=[pltpu.VMEM((2 * G, 16), jnp.float32),
                      pltpu.VMEM((2 * G, 1), jnp.float32)],
  )(agg2, degi, b2, n2g, Wf, bf)


def kernel(x_left, x_right, edge_index_left, edge_index_right,
           node2graph_left, node2graph_right, W1, b1, W2, b2, Wf, bf):
  f32 = jnp.float32
  srcl = edge_index_left[0].astype(jnp.int32).reshape(ECH, CHUNK)
  dstl = edge_index_left[1].astype(jnp.int32).reshape(ECH, CHUNK)
  srcr = edge_index_right[0].astype(jnp.int32).reshape(ECH, CHUNK)
  dstr = edge_index_right[1].astype(jnp.int32).reshape(ECH, CHUNK)
  zeros1 = jnp.zeros((OWN,), f32)
  ones1 = jnp.ones((CHUNK,), f32)
  zeros32 = jnp.zeros((OWN, 32), f32)
  zeros16 = jnp.zeros((OWN, 16), f32)

  dego, degi = _degree_hist(srcl, dstl, srcr, dstr, zeros1, ones1)
  dego_cat = dego.reshape(2 * N, 1)
  degi_cat = degi.reshape(2 * N, 1)

  x_cat = jnp.concatenate([x_left, x_right], axis=0)
  y1 = _proj_scale(x_cat, dego_cat, W1)
  agg1 = _edge_agg32(y1.reshape(NC, N, 32), srcl, dstl, srcr, dstr,
                     zeros32)

  y2 = _post_proj_scale(agg1.reshape(2 * N, 32), degi_cat, dego_cat,
                        b1.reshape(1, 32), W2)
  agg2 = _edge_agg16(y2.reshape(NC, N, 16), srcl, dstl, srcr, dstr,
                     zeros16)

  n2g_cat = jnp.concatenate([node2graph_left, node2graph_right]
                            ).astype(jnp.int32).reshape(2 * N, 1)
  return _readout_head(agg2.reshape(2 * N, 16), degi_cat,
                       b2.reshape(1, 16), n2g_cat, Wf, bf.reshape(1, 1))


# speedup vs baseline: 57.4770x; 1.1333x over previous
import functools

import jax
import jax.numpy as jnp
from jax import lax
from jax.experimental import pallas as pl
from jax.experimental.pallas import tpu as pltpu
from jax.experimental.pallas import tpu_sc as plsc

N = 10000
E = 320000
G = 64
NC, NS = 2, 16
OWN = 640
OWN_LAST = N - OWN * (NS - 1)
CHUNK = 128
ECH = E // CHUNK
NCH = ECH // NS
XTRA = ECH - NCH * NS
NBUF = 12
GAH = 6
NB = 2
BLK = N // NB

_MESH = plsc.VectorSubcoreMesh(
    core_axis_name="c", subcore_axis_name="s", num_cores=NC, num_subcores=NS)


def _per_tile_rows(t, fn):
  @pl.when(t < NS - 1)
  def _():
    fn(t * OWN, OWN)

  @pl.when(t == NS - 1)
  def _():
    fn((NS - 1) * OWN, OWN_LAST)


def _stage_idx(c, t, e3l, e3r, idx_v):
  s0 = t * NCH + jnp.minimum(t, XTRA)

  def cp(e3):
    @pl.when(t < XTRA)
    def _():
      pltpu.sync_copy(e3.at[pl.ds(s0, NCH + 1)], idx_v)

    @pl.when(t >= XTRA)
    def _():
      pltpu.sync_copy(e3.at[pl.ds(s0, NCH)], idx_v.at[pl.ds(0, NCH)])

  @pl.when(c == 0)
  def _():
    cp(e3l)

  @pl.when(c == 1)
  def _():
    cp(e3r)


@functools.partial(
    pl.kernel,
    out_type=(jax.ShapeDtypeStruct((NC, N), jnp.float32),
              jax.ShapeDtypeStruct((NC, N), jnp.float32)),
    mesh=_MESH,
    scratch_types=[
        pltpu.VMEM_SHARED((N,), jnp.float32),
        pltpu.VMEM_SHARED((N,), jnp.float32),
        pltpu.VMEM((NCH + 1, 2, CHUNK), jnp.int32),
        pltpu.VMEM((CHUNK,), jnp.float32),
        pltpu.SemaphoreType.DMA,
    ],
    compiler_params=pltpu.CompilerParams(use_tc_tiling_on_sc=False),
)
def _degree_hist(e3l, e3r, zero_hbm, one_hbm, dego_hbm, degi_hbm,
                 ho_sp, hi_sp, idx_v, ones_v, sem):
  c = lax.axis_index("c")
  t = lax.axis_index("s")

  def zero(r0, sz):
    pltpu.sync_copy(zero_hbm.at[pl.ds(0, sz)], ho_sp.at[pl.ds(r0, sz)])
    pltpu.sync_copy(zero_hbm.at[pl.ds(0, sz)], hi_sp.at[pl.ds(r0, sz)])

  _per_tile_rows(t, zero)
  pltpu.sync_copy(one_hbm, ones_v)
  _stage_idx(c, t, e3l, e3r, idx_v)
  plsc.subcore_barrier()

  K = 12

  def rounds(r, sp):
    def body(i, _):
      for b in range(K):
        pltpu.async_copy(ones_v, sp.at[idx_v.at[i * K + b, r]], sem, add=True)
      for b in range(K):
        pltpu.make_async_copy(ones_v, sp.at[idx_v.at[i * K + b, r]],
                              sem).wait()
      return 0
    lax.fori_loop(0, NCH // K, body, 0)

  rounds(0, ho_sp)
  rounds(1, hi_sp)

  @pl.when(t < XTRA)
  def _():
    pltpu.sync_copy(ones_v, ho_sp.at[idx_v.at[NCH, 0]], add=True)
    pltpu.sync_copy(ones_v, hi_sp.at[idx_v.at[NCH, 1]], add=True)

  plsc.subcore_barrier()

  def flush(r0, sz):
    pltpu.sync_copy(ho_sp.at[pl.ds(r0, sz)], dego_hbm.at[c].at[pl.ds(r0, sz)])
    pltpu.sync_copy(hi_sp.at[pl.ds(r0, sz)], degi_hbm.at[c].at[pl.ds(r0, sz)])

  _per_tile_rows(t, flush)


def _make_edge_agg(F):
  @functools.partial(
      pl.kernel,
      out_type=jax.ShapeDtypeStruct((NC, N, F), jnp.float32),
      mesh=_MESH,
      scratch_types=[
          pltpu.VMEM_SHARED((N, F), jnp.float32),
          pltpu.VMEM((NCH + 1, 2, CHUNK), jnp.int32),
          pltpu.VMEM((NBUF, CHUNK, F), jnp.float32),
          pltpu.SemaphoreType.DMA,
          pltpu.SemaphoreType.DMA((NBUF,)),
      ],
      compiler_params=pltpu.CompilerParams(use_tc_tiling_on_sc=False),
  )
  def agg(tbl_hbm, e3l, e3r, zero_hbm, out_hbm,
          acc_sp, idx_v, rows_v, gsem, ssem):
    c = lax.axis_index("c")
    t = lax.axis_index("s")

    def zero(r0, sz):
      pltpu.sync_copy(zero_hbm.at[pl.ds(0, sz)], acc_sp.at[pl.ds(r0, sz)])

    _per_tile_rows(t, zero)
    _stage_idx(c, t, e3l, e3r, idx_v)
    plsc.subcore_barrier()

    tbl = tbl_hbm.at[c]

    def fire_gather(j, s):
      pltpu.async_copy(tbl.at[idx_v.at[j, 0]], rows_v.at[s], gsem)

    def wait_gather(j, s):
      pltpu.make_async_copy(tbl.at[idx_v.at[j, 0]], rows_v.at[s], gsem).wait()

    def fire_scatter(j, s):
      pltpu.async_copy(rows_v.at[s], acc_sp.at[idx_v.at[j, 1]], ssem.at[s],
                       add=True)

    def wait_scatter(j, s):
      pltpu.make_async_copy(rows_v.at[s], acc_sp.at[idx_v.at[j, 1]],
                            ssem.at[s]).wait()

    for s in range(GAH):
      fire_gather(s, s)

    def body(i, _):
      j0 = i * NBUF
      for s in range(NBUF):
        j = j0 + s
        wait_gather(j, s)
        fire_scatter(j, s)
        f = j + GAH
        sf = (s + GAH) % NBUF

        @pl.when(f < NCH)
        def _():
          @pl.when(j >= GAH)
          def _():
            wait_scatter(j - GAH, sf)
          fire_gather(f, sf)
      return 0

    lax.fori_loop(0, NCH // NBUF, body, 0)
    for s in range(NBUF):
      wait_scatter(NCH - NBUF + s, s)

    @pl.when(t < XTRA)
    def _():
      pltpu.sync_copy(tbl.at[idx_v.at[NCH, 0]], rows_v.at[0])
      pltpu.sync_copy(rows_v.at[0], acc_sp.at[idx_v.at[NCH, 1]], add=True)

    plsc.subcore_barrier()

    def flush(r0, sz):
      pltpu.sync_copy(acc_sp.at[pl.ds(r0, sz)],
                      out_hbm.at[c].at[pl.ds(r0, sz)])

    _per_tile_rows(t, flush)

  return agg


_edge_agg32 = _make_edge_agg(32)
_edge_agg16 = _make_edge_agg(16)


def _proj_scale(x, deg, W):
  din, dout = W.shape

  def body(x_ref, d_ref, w_ref, o_ref):
    a = lax.rsqrt(jnp.maximum(d_ref[...], 1.0))
    o_ref[...] = jnp.dot(x_ref[...], w_ref[...],
                         preferred_element_type=jnp.float32) * a

  return pl.pallas_call(
      body,
      grid=(2 * NB,),
      in_specs=[pl.BlockSpec((BLK, din), lambda i: (i, 0)),
                pl.BlockSpec((BLK, 1), lambda i: (i, 0)),
                pl.BlockSpec((din, dout), lambda i: (0, 0))],
      out_specs=pl.BlockSpec((BLK, dout), lambda i: (i, 0)),
      out_shape=jax.ShapeDtypeStruct((2 * N, dout), jnp.float32),
  )(x, deg, W)


def _post_proj_scale(agg, degi, dego, b, W):
  din, dout = W.shape

  def body(a_ref, di_ref, do_ref, b_ref, w_ref, o_ref):
    bi = lax.rsqrt(jnp.maximum(di_ref[...], 1.0))
    ao = lax.rsqrt(jnp.maximum(do_ref[...], 1.0))
    h = jnp.maximum(a_ref[...] * bi + b_ref[...], 0.0)
    o_ref[...] = jnp.dot(h, w_ref[...],
                         preferred_element_type=jnp.float32) * ao

  return pl.pallas_call(
      body,
      grid=(2 * NB,),
      in_specs=[pl.BlockSpec((BLK, din), lambda i: (i, 0)),
                pl.BlockSpec((BLK, 1), lambda i: (i, 0)),
                pl.BlockSpec((BLK, 1), lambda i: (i, 0)),
                pl.BlockSpec((1, din), lambda i: (0, 0)),
                pl.BlockSpec((din, dout), lambda i: (0, 0))],
      out_specs=pl.BlockSpec((BLK, dout), lambda i: (i, 0)),
      out_shape=jax.ShapeDtypeStruct((2 * N, dout), jnp.float32),
  )(agg, degi, dego, b, W)


def _readout_head(agg2, degi, b2, n2g, Wf, bf):
  def body(a_ref, di_ref, b_ref, g_ref, wf_ref, bf_ref, o_ref, r_acc, c_acc):
    i = pl.program_id(0)

    @pl.when(i == 0)
    def _():
      r_acc[...] = jnp.zeros_like(r_acc)
      c_acc[...] = jnp.zeros_like(c_acc)

    bi = lax.rsqrt(jnp.maximum(di_ref[...], 1.0))
    h = jnp.maximum(a_ref[...] * bi + b_ref[...], 0.0)
    gid = g_ref[...] + (i // NB) * G
    onehot = (gid == lax.broadcasted_iota(jnp.int32, (BLK, 2 * G), 1)
              ).astype(jnp.float32)
    r_acc[...] += lax.dot_general(onehot, h, (((0,), (0,)), ((), ())),
                                  preferred_element_type=jnp.float32)
    c_acc[...] += jnp.sum(onehot, axis=0)[:, None]

    @pl.when(i == 2 * NB - 1)
    def _():
      m = r_acc[...] / jnp.maximum(c_acc[...], 1.0)
      hidden = m[:G] + m[G:]
      o_ref[...] = jax.nn.sigmoid(
          jnp.dot(hidden, wf_ref[...], preferred_element_type=jnp.float32)
          + bf_ref[...])

  return pl.pallas_call(
      body,
      grid=(2 * NB,),
      in_specs=[pl.BlockSpec((BLK, 16), lambda i: (i, 0)),
                pl.BlockSpec((BLK, 1), lambda i: (i, 0)),
                pl.BlockSpec((1, 16), lambda i: (0, 0)),
                pl.BlockSpec((BLK, 1), lambda i: (i, 0)),
                pl.BlockSpec((16, 1), lambda i: (0, 0)),
                pl.BlockSpec((1, 1), lambda i: (0, 0))],
      out_specs=pl.BlockSpec((G, 1), lambda i: (0, 0)),
      out_shape=jax.ShapeDtypeStruct((G, 1), jnp.float32),
      scratch_shapes=[pltpu.VMEM((2 * G, 16), jnp.float32),
                      pltpu.VMEM((2 * G, 1), jnp.float32)],
  )(agg2, degi, b2, n2g, Wf, bf)


def kernel(x_left, x_right, edge_index_left, edge_index_right,
           node2graph_left, node2graph_right, W1, b1, W2, b2, Wf, bf):
  f32 = jnp.float32
  e3l = (edge_index_left.astype(jnp.int32)
         .reshape(2, ECH, CHUNK).transpose(1, 0, 2))
  e3r = (edge_index_right.astype(jnp.int32)
         .reshape(2, ECH, CHUNK).transpose(1, 0, 2))
  zeros1 = jnp.zeros((OWN,), f32)
  ones1 = jnp.ones((CHUNK,), f32)
  zeros32 = jnp.zeros((OWN, 32), f32)
  zeros16 = jnp.zeros((OWN, 16), f32)

  dego, degi = _degree_hist(e3l, e3r, zeros1, ones1)
  dego_cat = dego.reshape(2 * N, 1)
  degi_cat = degi.reshape(2 * N, 1)

  x_cat = jnp.concatenate([x_left, x_right], axis=0)
  y1 = _proj_scale(x_cat, dego_cat, W1)
  agg1 = _edge_agg32(y1.reshape(NC, N, 32), e3l, e3r, zeros32)

  y2 = _post_proj_scale(agg1.reshape(2 * N, 32), degi_cat, dego_cat,
                        b1.reshape(1, 32), W2)
  agg2 = _edge_agg16(y2.reshape(NC, N, 16), e3l, e3r, zeros16)

  n2g_cat = jnp.concatenate([node2graph_left, node2graph_right]
                            ).astype(jnp.int32).reshape(2 * N, 1)
  return _readout_head(agg2.reshape(2 * N, 16), degi_cat,
                       b2.reshape(1, 16), n2g_cat, Wf, bf.reshape(1, 1))


# speedup vs baseline: 61.0587x; 1.0623x over previous
import functools

import jax
import jax.numpy as jnp
from jax import lax
from jax.experimental import pallas as pl
from jax.experimental.pallas import tpu as pltpu
from jax.experimental.pallas import tpu_sc as plsc

N = 10000
E = 320000
G = 64
NC, NS = 2, 16
OWN = 640
OWN_LAST = N - OWN * (NS - 1)
CHUNK = 128
ECH = E // CHUNK
NCH = ECH // NS
XTRA = ECH - NCH * NS
NB = 2
BLK = N // NB

_MESH = plsc.VectorSubcoreMesh(
    core_axis_name="c", subcore_axis_name="s", num_cores=NC, num_subcores=NS)


def _per_tile_rows(t, fn):
  @pl.when(t < NS - 1)
  def _():
    fn(t * OWN, OWN)

  @pl.when(t == NS - 1)
  def _():
    fn((NS - 1) * OWN, OWN_LAST)


def _stage_idx(c, t, e3l, e3r, idx_v):
  s0 = t * NCH + jnp.minimum(t, XTRA)

  def cp(e3):
    @pl.when(t < XTRA)
    def _():
      pltpu.sync_copy(e3.at[pl.ds(s0, NCH + 1)], idx_v)

    @pl.when(t >= XTRA)
    def _():
      pltpu.sync_copy(e3.at[pl.ds(s0, NCH)], idx_v.at[pl.ds(0, NCH)])

  @pl.when(c == 0)
  def _():
    cp(e3l)

  @pl.when(c == 1)
  def _():
    cp(e3r)


@functools.partial(
    pl.kernel,
    out_type=(jax.ShapeDtypeStruct((NC, N), jnp.float32),
              jax.ShapeDtypeStruct((NC, N), jnp.float32)),
    mesh=_MESH,
    scratch_types=[
        pltpu.VMEM_SHARED((N,), jnp.float32),
        pltpu.VMEM_SHARED((N,), jnp.float32),
        pltpu.VMEM((NCH + 1, 2, CHUNK), jnp.int32),
        pltpu.VMEM((CHUNK,), jnp.float32),
        pltpu.SemaphoreType.DMA,
    ],
    compiler_params=pltpu.CompilerParams(use_tc_tiling_on_sc=False),
)
def _degree_hist(e3l, e3r, zero_hbm, one_hbm, dego_hbm, degi_hbm,
                 ho_sp, hi_sp, idx_v, ones_v, sem):
  c = lax.axis_index("c")
  t = lax.axis_index("s")

  def zero(r0, sz):
    pltpu.sync_copy(zero_hbm.at[pl.ds(0, sz)], ho_sp.at[pl.ds(r0, sz)])
    pltpu.sync_copy(zero_hbm.at[pl.ds(0, sz)], hi_sp.at[pl.ds(r0, sz)])

  _per_tile_rows(t, zero)
  pltpu.sync_copy(one_hbm, ones_v)
  _stage_idx(c, t, e3l, e3r, idx_v)
  plsc.subcore_barrier()

  K = 12

  def rounds(r, sp):
    def body(i, _):
      for b in range(K):
        pltpu.async_copy(ones_v, sp.at[idx_v.at[i * K + b, r]], sem, add=True)
      for b in range(K):
        pltpu.make_async_copy(ones_v, sp.at[idx_v.at[i * K + b, r]],
                              sem).wait()
      return 0
    lax.fori_loop(0, NCH // K, body, 0)

  rounds(0, ho_sp)
  rounds(1, hi_sp)

  @pl.when(t < XTRA)
  def _():
    pltpu.sync_copy(ones_v, ho_sp.at[idx_v.at[NCH, 0]], add=True)
    pltpu.sync_copy(ones_v, hi_sp.at[idx_v.at[NCH, 1]], add=True)

  plsc.subcore_barrier()

  def flush(r0, sz):
    pltpu.sync_copy(ho_sp.at[pl.ds(r0, sz)], dego_hbm.at[c].at[pl.ds(r0, sz)])
    pltpu.sync_copy(hi_sp.at[pl.ds(r0, sz)], degi_hbm.at[c].at[pl.ds(r0, sz)])

  _per_tile_rows(t, flush)


def _make_edge_agg(F, NBUF, GAH):
  @functools.partial(
      pl.kernel,
      out_type=jax.ShapeDtypeStruct((NC, N, F), jnp.float32),
      mesh=_MESH,
      scratch_types=[
          pltpu.VMEM_SHARED((N, F), jnp.float32),
          pltpu.VMEM((NCH + 1, 2, CHUNK), jnp.int32),
          pltpu.VMEM((NBUF, CHUNK, F), jnp.float32),
          pltpu.SemaphoreType.DMA,
          pltpu.SemaphoreType.DMA((NBUF,)),
      ],
      compiler_params=pltpu.CompilerParams(use_tc_tiling_on_sc=False),
  )
  def agg(tbl_hbm, e3l, e3r, zero_hbm, out_hbm,
          acc_sp, idx_v, rows_v, gsem, ssem):
    c = lax.axis_index("c")
    t = lax.axis_index("s")

    def zero(r0, sz):
      pltpu.sync_copy(zero_hbm.at[pl.ds(0, sz)], acc_sp.at[pl.ds(r0, sz)])

    _per_tile_rows(t, zero)
    _stage_idx(c, t, e3l, e3r, idx_v)
    plsc.subcore_barrier()

    tbl = tbl_hbm.at[c]

    def fire_gather(j, s):
      pltpu.async_copy(tbl.at[idx_v.at[j, 0]], rows_v.at[s], gsem)

    def wait_gather(j, s):
      pltpu.make_async_copy(tbl.at[idx_v.at[j, 0]], rows_v.at[s], gsem).wait()

    def fire_scatter(j, s):
      pltpu.async_copy(rows_v.at[s], acc_sp.at[idx_v.at[j, 1]], ssem.at[s],
                       add=True)

    def wait_scatter(j, s):
      pltpu.make_async_copy(rows_v.at[s], acc_sp.at[idx_v.at[j, 1]],
                            ssem.at[s]).wait()

    for s in range(GAH):
      fire_gather(s, s)

    def body(i, _):
      j0 = i * NBUF
      for s in range(NBUF):
        j = j0 + s
        wait_gather(j, s)
        fire_scatter(j, s)
        f = j + GAH
        sf = (s + GAH) % NBUF

        @pl.when(f < NCH)
        def _():
          @pl.when(j >= NBUF - GAH)
          def _():
            wait_scatter(j + GAH - NBUF, sf)
          fire_gather(f, sf)
      return 0

    lax.fori_loop(0, NCH // NBUF, body, 0)
    for s in range(NBUF):
      wait_scatter(NCH - NBUF + s, s)

    @pl.when(t < XTRA)
    def _():
      pltpu.sync_copy(tbl.at[idx_v.at[NCH, 0]], rows_v.at[0])
      pltpu.sync_copy(rows_v.at[0], acc_sp.at[idx_v.at[NCH, 1]], add=True)

    plsc.subcore_barrier()

    def flush(r0, sz):
      pltpu.sync_copy(acc_sp.at[pl.ds(r0, sz)],
                      out_hbm.at[c].at[pl.ds(r0, sz)])

    _per_tile_rows(t, flush)

  return agg


_edge_agg32 = _make_edge_agg(32, 13, 6)
_edge_agg16 = _make_edge_agg(16, 26, 13)


def _proj_scale(x, deg, W):
  din, dout = W.shape

  def body(x_ref, d_ref, w_ref, o_ref):
    a = lax.rsqrt(jnp.maximum(d_ref[...], 1.0))
    o_ref[...] = jnp.dot(x_ref[...], w_ref[...],
                         preferred_element_type=jnp.float32) * a

  return pl.pallas_call(
      body,
      grid=(2 * NB,),
      in_specs=[pl.BlockSpec((BLK, din), lambda i: (i, 0)),
                pl.BlockSpec((BLK, 1), lambda i: (i, 0)),
                pl.BlockSpec((din, dout), lambda i: (0, 0))],
      out_specs=pl.BlockSpec((BLK, dout), lambda i: (i, 0)),
      out_shape=jax.ShapeDtypeStruct((2 * N, dout), jnp.float32),
  )(x, deg, W)


def _post_proj_scale(agg, degi, dego, b, W):
  din, dout = W.shape

  def body(a_ref, di_ref, do_ref, b_ref, w_ref, o_ref):
    bi = lax.rsqrt(jnp.maximum(di_ref[...], 1.0))
    ao = lax.rsqrt(jnp.maximum(do_ref[...], 1.0))
    h = jnp.maximum(a_ref[...] * bi + b_ref[...], 0.0)
    o_ref[...] = jnp.dot(h, w_ref[...],
                         preferred_element_type=jnp.float32) * ao

  return pl.pallas_call(
      body,
      grid=(2 * NB,),
      in_specs=[pl.BlockSpec((BLK, din), lambda i: (i, 0)),
                pl.BlockSpec((BLK, 1), lambda i: (i, 0)),
                pl.BlockSpec((BLK, 1), lambda i: (i, 0)),
                pl.BlockSpec((1, din), lambda i: (0, 0)),
                pl.BlockSpec((din, dout), lambda i: (0, 0))],
      out_specs=pl.BlockSpec((BLK, dout), lambda i: (i, 0)),
      out_shape=jax.ShapeDtypeStruct((2 * N, dout), jnp.float32),
  )(agg, degi, dego, b, W)


def _readout_head(agg2, degi, b2, n2g, Wf, bf):
  def body(a_ref, di_ref, b_ref, g_ref, wf_ref, bf_ref, o_ref, r_acc, c_acc):
    i = pl.program_id(0)

    @pl.when(i == 0)
    def _():
      r_acc[...] = jnp.zeros_like(r_acc)
      c_acc[...] = jnp.zeros_like(c_acc)

    bi = lax.rsqrt(jnp.maximum(di_ref[...], 1.0))
    h = jnp.maximum(a_ref[...] * bi + b_ref[...], 0.0)
    gid = g_ref[...] + (i // NB) * G
    onehot = (gid == lax.broadcasted_iota(jnp.int32, (BLK, 2 * G), 1)
              ).astype(jnp.float32)
    r_acc[...] += lax.dot_general(onehot, h, (((0,), (0,)), ((), ())),
                                  preferred_element_type=jnp.float32)
    c_acc[...] += jnp.sum(onehot, axis=0)[:, None]

    @pl.when(i == 2 * NB - 1)
    def _():
      m = r_acc[...] / jnp.maximum(c_acc[...], 1.0)
      hidden = m[:G] + m[G:]
      o_ref[...] = jax.nn.sigmoid(
          jnp.dot(hidden, wf_ref[...], preferred_element_type=jnp.float32)
          + bf_ref[...])

  return pl.pallas_call(
      body,
      grid=(2 * NB,),
      in_specs=[pl.BlockSpec((BLK, 16), lambda i: (i, 0)),
                pl.BlockSpec((BLK, 1), lambda i: (i, 0)),
                pl.BlockSpec((1, 16), lambda i: (0, 0)),
                pl.BlockSpec((BLK, 1), lambda i: (i, 0)),
                pl.BlockSpec((16, 1), lambda i: (0, 0)),
                pl.BlockSpec((1, 1), lambda i: (0, 0))],
      out_specs=pl.BlockSpec((G, 1), lambda i: (0, 0)),
      out_shape=jax.ShapeDtypeStruct((G, 1), jnp.float32),
      scratch_shapes=[pltpu.VMEM((2 * G, 16), jnp.float32),
                      pltpu.VMEM((2 * G, 1), jnp.float32)],
  )(agg2, degi, b2, n2g, Wf, bf)


def kernel(x_left, x_right, edge_index_left, edge_index_right,
           node2graph_left, node2graph_right, W1, b1, W2, b2, Wf, bf):
  f32 = jnp.float32
  e3l = (edge_index_left.astype(jnp.int32)
         .reshape(2, ECH, CHUNK).transpose(1, 0, 2))
  e3r = (edge_index_right.astype(jnp.int32)
         .reshape(2, ECH, CHUNK).transpose(1, 0, 2))
  zeros1 = jnp.zeros((OWN,), f32)
  ones1 = jnp.ones((CHUNK,), f32)
  zeros32 = jnp.zeros((OWN, 32), f32)
  zeros16 = jnp.zeros((OWN, 16), f32)

  dego, degi = _degree_hist(e3l, e3r, zeros1, ones1)
  dego_cat = dego.reshape(2 * N, 1)
  degi_cat = degi.reshape(2 * N, 1)

  x_cat = jnp.concatenate([x_left, x_right], axis=0)
  y1 = _proj_scale(x_cat, dego_cat, W1)
  agg1 = _edge_agg32(y1.reshape(NC, N, 32), e3l, e3r, zeros32)

  y2 = _post_proj_scale(agg1.reshape(2 * N, 32), degi_cat, dego_cat,
                        b1.reshape(1, 32), W2)
  agg2 = _edge_agg16(y2.reshape(NC, N, 16), e3l, e3r, zeros16)

  n2g_cat = jnp.concatenate([node2graph_left, node2graph_right]
                            ).astype(jnp.int32).reshape(2 * N, 1)
  return _readout_head(agg2.reshape(2 * N, 16), degi_cat,
                       b2.reshape(1, 16), n2g_cat, Wf, bf.reshape(1, 1))
